# baseline (speedup 1.0000x reference)
import dataclasses
import functools

import jax
import jax.numpy as jnp
import numpy as np
from jax import lax
from jax.experimental import pallas as pl
from jax.experimental.pallas import tpu as pltpu
from jax.experimental.pallas import tpu_sc as plsc

B, C, H, W = 16, 17, 384, 384
NKPT = B * 8 * C
NC, NS, L = 2, 16, 16
NW = NC * NS
KTC = 9
CHS = C - KTC
NSLAB = B * CHS
JMAX = NSLAB // NW
PER_W = 80
NPAD = NW * PER_W
KCH = PER_W // L
RPC = 64
NCHK = H // RPC
SPR = W // L
UNR = 8
PAF = 0.001
_NEG = -3.0e38


def _sc_body(pred_hbm, xs_hbm, ys_hbm, rb_hbm,
             m_out, s_out, g_out,
             xs_v, ys_v, rb_v, idx_v, rows_v,
             bufA, bufB, mvec, svec, gvec, macc_v, sacc_v, tmax_v,
             semA, semB, semG):
    cid = lax.axis_index("c")
    sid = lax.axis_index("s")
    w = sid * NC + cid
    lane = lax.iota(jnp.int32, L)

    kbase = w * PER_W
    pltpu.sync_copy(xs_hbm.at[pl.ds(kbase, PER_W)], xs_v)
    pltpu.sync_copy(ys_hbm.at[pl.ds(kbase, PER_W)], ys_v)
    pltpu.sync_copy(rb_hbm.at[pl.ds(kbase, PER_W)], rb_v)
    for j in range(KCH):
        sl = pl.ds(j * L, L)
        idx_v[sl] = rb_v[sl] + ys_v[sl]
    gdma = pltpu.make_async_copy(pred_hbm.at[idx_v], rows_v, semG)
    gdma.start()

    mvec[...] = jnp.full((L,), _NEG, jnp.float32)
    svec[...] = jnp.zeros((L,), jnp.float32)

    def _start(row0, q, buf, sem):
        pltpu.make_async_copy(
            pred_hbm.at[pl.ds(row0 + q * RPC, RPC)], buf, sem).start()

    def _wait(buf, sem):
        pltpu.make_async_copy(
            pred_hbm.at[pl.ds(0, RPC)], buf, sem).wait()

    def _p2(buf, shift):
        def p2(r, ss):
            ss = list(ss)
            for k in range(SPR):
                ss[k % UNR] = ss[k % UNR] + jnp.exp(
                    buf[r, pl.ds(k * L, L)] - shift)
            return tuple(ss)

        s_u = lax.fori_loop(
            0, RPC, p2,
            tuple(jnp.zeros((L,), jnp.float32) for _ in range(UNR)))
        return functools.reduce(jnp.add, s_u)

    def _chunk_fused(buf):
        M = macc_v[...]

        def pf(r, carry):
            ms = list(carry[:UNR])
            ss = list(carry[UNR:])
            for k in range(SPR):
                x = buf[r, pl.ds(k * L, L)]
                ms[k % UNR] = jnp.maximum(ms[k % UNR], x)
                ss[k % UNR] = ss[k % UNR] + jnp.exp(x - M)
            return tuple(ms) + tuple(ss)

        init = (tuple(jnp.full((L,), _NEG, jnp.float32) for _ in range(UNR))
                + tuple(jnp.zeros((L,), jnp.float32) for _ in range(UNR)))
        res = lax.fori_loop(0, RPC, pf, init)
        mq = functools.reduce(jnp.maximum, res[:UNR])
        sq = functools.reduce(jnp.add, res[UNR:])
        amax = jnp.max(jnp.abs(mq - M))

        @pl.when(amax <= 60.0)
        def _():
            sacc_v[...] = sacc_v[...] + sq

        @pl.when(amax > 60.0)
        def _():
            sacc_v[...] = (sacc_v[...] * jnp.exp(M - mq)) + _p2(buf, mq)
            macc_v[...] = mq

    macc_v[...] = jnp.full((L,), _NEG, jnp.float32)

    def _slab(j, _):
        s_id = w + NW * j
        b = s_id // CHS
        c = KTC + s_id - b * CHS
        row0 = (b * C + c) * H
        sacc_v[...] = jnp.zeros((L,), jnp.float32)
        _start(row0, 0, bufA, semA)

        def pair(g2, _):
            q0 = 2 * g2
            _start(row0, q0 + 1, bufB, semB)
            _wait(bufA, semA)
            _chunk_fused(bufA)

            @pl.when(q0 + 2 < NCHK)
            def _():
                _start(row0, q0 + 2, bufA, semA)

            _wait(bufB, semB)
            _chunk_fused(bufB)
            return 0

        lax.fori_loop(0, NCHK // 2, pair, 0)

        m_sl = jnp.max(macc_v[...])
        s_sl = jnp.sum(sacc_v[...] * jnp.exp(macc_v[...] - m_sl))
        mvec[...] = jnp.where(lane == j, m_sl, mvec[...])
        svec[...] = jnp.where(lane == j, s_sl, svec[...])
        return 0

    lax.fori_loop(0, JMAX, _slab, 0)

    gdma.wait()
    gacc = jnp.zeros((L,), jnp.float32)
    for j in range(KCH):
        k = kbase + j * L + lane
        rloc = j * L + lane
        cols = xs_v[pl.ds(j * L, L)]
        v = plsc.load_gather(rows_v, [rloc, cols])
        gacc = gacc + jnp.where(k < NKPT, v, jnp.zeros((L,), jnp.float32))
    gvec[...] = gacc

    pltpu.sync_copy(mvec, m_out.at[pl.ds(w * L, L)])
    pltpu.sync_copy(svec, s_out.at[pl.ds(w * L, L)])
    pltpu.sync_copy(gvec, g_out.at[pl.ds(w * L, L)])


@functools.lru_cache(maxsize=1)
def _sc_kernel():
    cp = pltpu.CompilerParams()
    if "needs_layout_passes" in pltpu.CompilerParams.__dataclass_fields__:
        cp = dataclasses.replace(cp, needs_layout_passes=False)
    return pl.kernel(
        _sc_body,
        compiler_params=cp,
        out_type=(jax.ShapeDtypeStruct((NW * L,), jnp.float32),
                  jax.ShapeDtypeStruct((NW * L,), jnp.float32),
                  jax.ShapeDtypeStruct((NW * L,), jnp.float32)),
        mesh=plsc.VectorSubcoreMesh(
            core_axis_name="c", subcore_axis_name="s",
            num_cores=NC, num_subcores=NS),
        scratch_types=[
            pltpu.VMEM((PER_W,), jnp.int32),
            pltpu.VMEM((PER_W,), jnp.int32),
            pltpu.VMEM((PER_W,), jnp.int32),
            pltpu.VMEM((PER_W,), jnp.int32),
            pltpu.VMEM((PER_W, W), jnp.float32),
            pltpu.VMEM((RPC, W), jnp.float32),
            pltpu.VMEM((RPC, W), jnp.float32),
            pltpu.VMEM((L,), jnp.float32),
            pltpu.VMEM((L,), jnp.float32),
            pltpu.VMEM((L,), jnp.float32),
            pltpu.VMEM((L,), jnp.float32),
            pltpu.VMEM((L,), jnp.float32),
            pltpu.VMEM((L,), jnp.float32),
            pltpu.SemaphoreType.DMA,
            pltpu.SemaphoreType.DMA,
            pltpu.SemaphoreType.DMA,
        ],
    )


def _tc_dense_body(pred_ref, out_ref, n_ref):
    c = pl.program_id(0)

    @pl.when(c == 0)
    def _():
        n_ref[...] = jnp.zeros((8, 128), jnp.float32)

    x = pred_ref[...].reshape(B * H, W)
    m_t = jnp.max(x)
    s_t = jnp.sum(jnp.exp(x - m_t))
    ones = jnp.ones((8, 128), jnp.float32)
    n_ref[...] = n_ref[...] + ones * m_t + jnp.log(ones * s_t)

    @pl.when(c == KTC - 1)
    def _():
        out_ref[...] = n_ref[...]


@jax.jit
def _tc_dense_call(predictions):
    return pl.pallas_call(
        _tc_dense_body,
        grid=(KTC,),
        in_specs=[pl.BlockSpec((B, 1, H, W), lambda c: (0, c, 0, 0))],
        out_specs=pl.BlockSpec((8, 128), lambda c: (0, 0)),
        out_shape=jax.ShapeDtypeStruct((8, 128), jnp.float32),
        scratch_shapes=[pltpu.VMEM((8, 128), jnp.float32)],
        compiler_params=pltpu.CompilerParams(
            dimension_semantics=("arbitrary",)),
    )(predictions)


def _combine_body(m_ref, s_ref, g_ref, cm_ref, ntc_ref, out_ref):
    m2 = m_ref[...]
    s2 = s_ref[...]
    nsum = ntc_ref[...]
    for ci in range(CHS):
        mask = cm_ref[ci]
        mvals = jnp.where(mask > 0.0, m2, _NEG)
        m_c = jnp.max(mvals)
        s_c = jnp.sum(jnp.where(mask > 0.0, s2 * jnp.exp(m2 - m_c), 0.0))
        nsum = nsum + m_c + jnp.log(jnp.full((8, 128), s_c))
    g = jnp.sum(g_ref[...])
    out_ref[...] = PAF * (nsum * (1.0 / C) - g * (1.0 / NKPT))


@jax.jit
def _combine_call(m_out, s_out, g_out, cmask, ntc):
    return pl.pallas_call(
        _combine_body,
        in_specs=[pl.BlockSpec(memory_space=pltpu.VMEM)] * 5,
        out_specs=pl.BlockSpec(memory_space=pltpu.VMEM),
        out_shape=jax.ShapeDtypeStruct((8, 128), jnp.float32),
    )(m_out.reshape(4, 128), s_out.reshape(4, 128),
      g_out.reshape(4, 128), cmask, ntc)


_K = np.arange(NPAD)
_RB_CONST = np.where(
    _K < NKPT, ((_K // (8 * C)) * C + _K % C) * H, 0).astype(np.int32)

_CMASK = np.zeros((CHS, NW * L), np.float32)
for _w in range(NW):
    for _j in range(JMAX):
        _t = _w + NW * _j
        if _t < NSLAB:
            _CMASK[_t % CHS, _w * L + _j] = 1.0
_CMASK = _CMASK.reshape(CHS, 4, 128)


def kernel(predictions, targets):
    t = targets.astype(jnp.int32)
    xs = jnp.pad(t[..., 0].reshape(-1), (0, NPAD - NKPT))
    ys = jnp.pad(t[..., 1].reshape(-1), (0, NPAD - NKPT))
    rb = jnp.asarray(_RB_CONST)
    pred2 = predictions.reshape(B * C * H, W)
    m_out, s_out, g_out = _sc_kernel()(pred2, xs, ys, rb)
    ntc = _tc_dense_call(predictions)
    out = _combine_call(m_out, s_out, g_out, jnp.asarray(_CMASK), ntc)
    return out[0, 0]


# speedup vs baseline: 1.1400x; 1.1400x over previous
import dataclasses
import functools

import jax
import jax.numpy as jnp
import numpy as np
from jax import lax
from jax.experimental import pallas as pl
from jax.experimental.pallas import tpu as pltpu
from jax.experimental.pallas import tpu_sc as plsc

B, C, H, W = 16, 17, 384, 384
NKPT = B * 8 * C
NC, NS, L = 2, 16, 16
NW = NC * NS
KTC = 12
CHS = C - KTC
NSLAB = B * CHS
JMAX = NSLAB // NW
PER_W = 80
NPAD = NW * PER_W
KCH = PER_W // L
RPC = 64
NCHK = H // RPC
SPR = W // L
UNR = 8
PAF = 0.001
_NEG = -3.0e38


def _sc_body(pred_hbm, xs_hbm, ys_hbm, rb_hbm,
             m_out, s_out, g_out,
             xs_v, ys_v, rb_v, idx_v, rows_v,
             bufA, bufB, mvec, svec, gvec, macc_v, sacc_v, tmax_v,
             semA, semB, semG):
    cid = lax.axis_index("c")
    sid = lax.axis_index("s")
    w = sid * NC + cid
    lane = lax.iota(jnp.int32, L)

    kbase = w * PER_W
    pltpu.sync_copy(xs_hbm.at[pl.ds(kbase, PER_W)], xs_v)
    pltpu.sync_copy(ys_hbm.at[pl.ds(kbase, PER_W)], ys_v)
    pltpu.sync_copy(rb_hbm.at[pl.ds(kbase, PER_W)], rb_v)
    for j in range(KCH):
        sl = pl.ds(j * L, L)
        idx_v[sl] = rb_v[sl] + ys_v[sl]
    gdma = pltpu.make_async_copy(pred_hbm.at[idx_v], rows_v, semG)
    gdma.start()

    mvec[...] = jnp.full((L,), _NEG, jnp.float32)
    svec[...] = jnp.zeros((L,), jnp.float32)

    def _start(row0, q, buf, sem):
        pltpu.make_async_copy(
            pred_hbm.at[pl.ds(row0 + q * RPC, RPC)], buf, sem).start()

    def _wait(buf, sem):
        pltpu.make_async_copy(
            pred_hbm.at[pl.ds(0, RPC)], buf, sem).wait()

    def _p2(buf, shift):
        def p2(r, ss):
            ss = list(ss)
            for k in range(SPR):
                ss[k % UNR] = ss[k % UNR] + jnp.exp(
                    buf[r, pl.ds(k * L, L)] - shift)
            return tuple(ss)

        s_u = lax.fori_loop(
            0, RPC, p2,
            tuple(jnp.zeros((L,), jnp.float32) for _ in range(UNR)))
        return functools.reduce(jnp.add, s_u)

    def _chunk_fused(buf):
        M = macc_v[...]

        def pf(r, carry):
            ms = list(carry[:UNR])
            ss = list(carry[UNR:])
            for k in range(SPR):
                x = buf[r, pl.ds(k * L, L)]
                ms[k % UNR] = jnp.maximum(ms[k % UNR], x)
                ss[k % UNR] = ss[k % UNR] + jnp.exp(x - M)
            return tuple(ms) + tuple(ss)

        init = (tuple(jnp.full((L,), _NEG, jnp.float32) for _ in range(UNR))
                + tuple(jnp.zeros((L,), jnp.float32) for _ in range(UNR)))
        res = lax.fori_loop(0, RPC, pf, init)
        mq = functools.reduce(jnp.maximum, res[:UNR])
        sq = functools.reduce(jnp.add, res[UNR:])
        amax = jnp.max(jnp.abs(mq - M))

        @pl.when(amax <= 60.0)
        def _():
            sacc_v[...] = sacc_v[...] + sq

        @pl.when(amax > 60.0)
        def _():
            sacc_v[...] = (sacc_v[...] * jnp.exp(M - mq)) + _p2(buf, mq)
            macc_v[...] = mq

    macc_v[...] = jnp.full((L,), _NEG, jnp.float32)

    def _slab(j, _):
        s_id = w + NW * j
        b = s_id // CHS
        c = KTC + s_id - b * CHS
        row0 = (b * C + c) * H
        sacc_v[...] = jnp.zeros((L,), jnp.float32)
        _start(row0, 0, bufA, semA)

        def pair(g2, _):
            q0 = 2 * g2
            _start(row0, q0 + 1, bufB, semB)
            _wait(bufA, semA)
            _chunk_fused(bufA)

            @pl.when(q0 + 2 < NCHK)
            def _():
                _start(row0, q0 + 2, bufA, semA)

            _wait(bufB, semB)
            _chunk_fused(bufB)
            return 0

        lax.fori_loop(0, NCHK // 2, pair, 0)

        m_sl = jnp.max(macc_v[...])
        s_sl = jnp.sum(sacc_v[...] * jnp.exp(macc_v[...] - m_sl))
        mvec[...] = jnp.where(lane == j, m_sl, mvec[...])
        svec[...] = jnp.where(lane == j, s_sl, svec[...])
        return 0

    lax.fori_loop(0, JMAX, _slab, 0)

    gdma.wait()
    gacc = jnp.zeros((L,), jnp.float32)
    for j in range(KCH):
        k = kbase + j * L + lane
        rloc = j * L + lane
        cols = xs_v[pl.ds(j * L, L)]
        v = plsc.load_gather(rows_v, [rloc, cols])
        gacc = gacc + jnp.where(k < NKPT, v, jnp.zeros((L,), jnp.float32))
    gvec[...] = gacc

    pltpu.sync_copy(mvec, m_out.at[pl.ds(w * L, L)])
    pltpu.sync_copy(svec, s_out.at[pl.ds(w * L, L)])
    pltpu.sync_copy(gvec, g_out.at[pl.ds(w * L, L)])


@functools.lru_cache(maxsize=1)
def _sc_kernel():
    cp = pltpu.CompilerParams()
    if "needs_layout_passes" in pltpu.CompilerParams.__dataclass_fields__:
        cp = dataclasses.replace(cp, needs_layout_passes=False)
    return pl.kernel(
        _sc_body,
        compiler_params=cp,
        out_type=(jax.ShapeDtypeStruct((NW * L,), jnp.float32),
                  jax.ShapeDtypeStruct((NW * L,), jnp.float32),
                  jax.ShapeDtypeStruct((NW * L,), jnp.float32)),
        mesh=plsc.VectorSubcoreMesh(
            core_axis_name="c", subcore_axis_name="s",
            num_cores=NC, num_subcores=NS),
        scratch_types=[
            pltpu.VMEM((PER_W,), jnp.int32),
            pltpu.VMEM((PER_W,), jnp.int32),
            pltpu.VMEM((PER_W,), jnp.int32),
            pltpu.VMEM((PER_W,), jnp.int32),
            pltpu.VMEM((PER_W, W), jnp.float32),
            pltpu.VMEM((RPC, W), jnp.float32),
            pltpu.VMEM((RPC, W), jnp.float32),
            pltpu.VMEM((L,), jnp.float32),
            pltpu.VMEM((L,), jnp.float32),
            pltpu.VMEM((L,), jnp.float32),
            pltpu.VMEM((L,), jnp.float32),
            pltpu.VMEM((L,), jnp.float32),
            pltpu.VMEM((L,), jnp.float32),
            pltpu.SemaphoreType.DMA,
            pltpu.SemaphoreType.DMA,
            pltpu.SemaphoreType.DMA,
        ],
    )


def _tc_dense_body(pred_ref, out_ref, n_ref):
    c = pl.program_id(0)

    @pl.when(c == 0)
    def _():
        n_ref[...] = jnp.zeros((8, 128), jnp.float32)

    x = pred_ref[...].reshape(B * H, W)
    m_t = jnp.max(x)
    s_t = jnp.sum(jnp.exp(x - m_t))
    ones = jnp.ones((8, 128), jnp.float32)
    n_ref[...] = n_ref[...] + ones * m_t + jnp.log(ones * s_t)

    @pl.when(c == KTC - 1)
    def _():
        out_ref[...] = n_ref[...]


@jax.jit
def _tc_dense_call(predictions):
    return pl.pallas_call(
        _tc_dense_body,
        grid=(KTC,),
        in_specs=[pl.BlockSpec((B, 1, H, W), lambda c: (0, c, 0, 0))],
        out_specs=pl.BlockSpec((8, 128), lambda c: (0, 0)),
        out_shape=jax.ShapeDtypeStruct((8, 128), jnp.float32),
        scratch_shapes=[pltpu.VMEM((8, 128), jnp.float32)],
        compiler_params=pltpu.CompilerParams(
            dimension_semantics=("arbitrary",)),
    )(predictions)


def _combine_body(m_ref, s_ref, g_ref, cm_ref, ntc_ref, out_ref):
    m2 = m_ref[...]
    s2 = s_ref[...]
    nsum = ntc_ref[...]
    for ci in range(CHS):
        mask = cm_ref[ci]
        mvals = jnp.where(mask > 0.0, m2, _NEG)
        m_c = jnp.max(mvals)
        s_c = jnp.sum(jnp.where(mask > 0.0, s2 * jnp.exp(m2 - m_c), 0.0))
        nsum = nsum + m_c + jnp.log(jnp.full((8, 128), s_c))
    g = jnp.sum(g_ref[...])
    out_ref[...] = PAF * (nsum * (1.0 / C) - g * (1.0 / NKPT))


@jax.jit
def _combine_call(m_out, s_out, g_out, cmask, ntc):
    return pl.pallas_call(
        _combine_body,
        in_specs=[pl.BlockSpec(memory_space=pltpu.VMEM)] * 5,
        out_specs=pl.BlockSpec(memory_space=pltpu.VMEM),
        out_shape=jax.ShapeDtypeStruct((8, 128), jnp.float32),
    )(m_out.reshape(4, 128), s_out.reshape(4, 128),
      g_out.reshape(4, 128), cmask, ntc)


_K = np.arange(NPAD)
_RB_CONST = np.where(
    _K < NKPT, ((_K // (8 * C)) * C + _K % C) * H, 0).astype(np.int32)

_CMASK = np.zeros((CHS, NW * L), np.float32)
for _w in range(NW):
    for _j in range(JMAX):
        _t = _w + NW * _j
        if _t < NSLAB:
            _CMASK[_t % CHS, _w * L + _j] = 1.0
_CMASK = _CMASK.reshape(CHS, 4, 128)


def kernel(predictions, targets):
    t = targets.astype(jnp.int32)
    xs = jnp.pad(t[..., 0].reshape(-1), (0, NPAD - NKPT))
    ys = jnp.pad(t[..., 1].reshape(-1), (0, NPAD - NKPT))
    rb = jnp.asarray(_RB_CONST)
    pred2 = predictions.reshape(B * C * H, W)
    m_out, s_out, g_out = _sc_kernel()(pred2, xs, ys, rb)
    ntc = _tc_dense_call(predictions)
    out = _combine_call(m_out, s_out, g_out, jnp.asarray(_CMASK), ntc)
    return out[0, 0]
